# baseline (device time: 25155 ns/iter reference)
import jax
import jax.numpy as jnp
from jax import lax
from jax.experimental import pallas as pl
from jax.experimental.pallas import tpu as pltpu

N_DEV = 32
N_STEPS = 5


def kernel(x):
    m, n = x.shape

    def body(x_ref, out_ref, tbuf, acc, sbuf, rbuf, send_sems, recv_sems, ack_sem):
        my = lax.axis_index("i")

        barrier_sem = pltpu.get_barrier_semaphore()
        for d in range(N_STEPS):
            s = 1 << d

            @pl.when(my + s < N_DEV)
            def _():
                pl.semaphore_signal(
                    barrier_sem,
                    inc=1,
                    device_id=(my + s,),
                    device_id_type=pl.DeviceIdType.MESH,
                )

            @pl.when(my >= s)
            def _():
                pl.semaphore_signal(
                    barrier_sem,
                    inc=1,
                    device_id=(my - s,),
                    device_id_type=pl.DeviceIdType.MESH,
                )

        for d in range(N_STEPS):
            s = 1 << d

            @pl.when(my + s < N_DEV)
            def _():
                pl.semaphore_wait(barrier_sem, 1)

            @pl.when(my >= s)
            def _():
                pl.semaphore_wait(barrier_sem, 1)

        out_ref[:, :] = x_ref[:, :]
        size = m
        while size > 1:
            half = size // 2
            out_ref[:half, :] = out_ref[:half, :] * out_ref[half:size, :]
            size = half
        tbuf[:, :] = out_ref[0:1, :]
        acc[:, :] = out_ref[0:1, :]

        cumprod_passes = []

        def _copy():
            out_ref[:, :] = x_ref[:, :]

        cumprod_passes.append(_copy)
        shift = 1
        while shift < m:

            def _pass(s=shift):
                out_ref[s:, :] = out_ref[s:, :] * out_ref[: m - s, :]

            cumprod_passes.append(_pass)
            shift *= 2
        per_step = max(1, len(cumprod_passes) // N_STEPS)

        send_descs = []
        unit = 0
        for d in range(N_STEPS):
            s = 1 << d
            sbuf[d, :, :] = acc[:, :]
            send = pltpu.make_async_remote_copy(
                src_ref=sbuf.at[d],
                dst_ref=rbuf.at[d],
                send_sem=send_sems.at[d],
                recv_sem=recv_sems.at[d],
                device_id=(my + s,),
                device_id_type=pl.DeviceIdType.MESH,
            )
            send_descs.append(send)

            @pl.when(my + s < N_DEV)
            def _():
                send.start()

            for _ in range(per_step):
                if unit < len(cumprod_passes):
                    cumprod_passes[unit]()
                    unit += 1

            @pl.when(my >= s)
            def _():
                send.wait_recv()
                acc[:, :] = acc[:, :] * rbuf[d, :, :]
                pl.semaphore_signal(
                    ack_sem,
                    inc=1,
                    device_id=(my - s,),
                    device_id_type=pl.DeviceIdType.MESH,
                )

        while unit < len(cumprod_passes):
            cumprod_passes[unit]()
            unit += 1

        for d in range(N_STEPS):
            s = 1 << d

            @pl.when(my + s < N_DEV)
            def _():
                send_descs[d].wait_send()
                pl.semaphore_wait(ack_sem, 1)

        acc[:, :] = acc[:, :] / tbuf[:, :]
        out_ref[:, :] = out_ref[:, :] * acc[:, :]

    return pl.pallas_call(
        body,
        out_shape=jax.ShapeDtypeStruct((m, n), jnp.float32),
        in_specs=[pl.BlockSpec(memory_space=pltpu.VMEM)],
        out_specs=pl.BlockSpec(memory_space=pltpu.VMEM),
        scratch_shapes=[
            pltpu.VMEM((1, n), jnp.float32),
            pltpu.VMEM((1, n), jnp.float32),
            pltpu.VMEM((N_STEPS, 1, n), jnp.float32),
            pltpu.VMEM((N_STEPS, 1, n), jnp.float32),
            pltpu.SemaphoreType.DMA((N_STEPS,)),
            pltpu.SemaphoreType.DMA((N_STEPS,)),
            pltpu.SemaphoreType.REGULAR,
        ],
        compiler_params=pltpu.CompilerParams(collective_id=0),
    )(x)


# device time: 24679 ns/iter; 1.0193x vs baseline; 1.0193x over previous
import jax
import jax.numpy as jnp
from jax import lax
from jax.experimental import pallas as pl
from jax.experimental.pallas import tpu as pltpu

N_DEV = 32
N_STEPS = 5
B = 8


def kernel(x):
    m, n = x.shape
    nb = m // B

    def body(
        x_ref, out_ref, tbuf, acc, s3, bt3, ebp3, sbuf, rbuf,
        send_sems, recv_sems, ack_sem,
    ):
        my = lax.axis_index("i")

        barrier_sem = pltpu.get_barrier_semaphore()
        for d in range(N_STEPS):
            s = 1 << d

            @pl.when(my + s < N_DEV)
            def _():
                pl.semaphore_signal(
                    barrier_sem, inc=1,
                    device_id=(my + s,), device_id_type=pl.DeviceIdType.MESH,
                )

            @pl.when(my >= s)
            def _():
                pl.semaphore_signal(
                    barrier_sem, inc=1,
                    device_id=(my - s,), device_id_type=pl.DeviceIdType.MESH,
                )

        for d in range(N_STEPS):
            s = 1 << d

            @pl.when(my + s < N_DEV)
            def _():
                pl.semaphore_wait(barrier_sem, 1)

            @pl.when(my >= s)
            def _():
                pl.semaphore_wait(barrier_sem, 1)

        out_ref[:, :] = x_ref[:, :]
        size = m
        while size > 1:
            half = size // 2
            out_ref[:half, :] = out_ref[:half, :] * out_ref[half:size, :]
            size = half
        tbuf[:, :] = out_ref[0:1, :]
        acc[:, :] = out_ref[0:1, :]

        def _u_copy_in():
            s3[:, :, :] = jnp.reshape(x_ref[:, :], (nb, B, n))

        def _u_hs(sh):
            def f():
                s3[:, sh:, :] = s3[:, sh:, :] * s3[:, : B - sh, :]

            return f

        def _u_block_scan():
            bt3[:, :, :] = s3[:, B - 1 : B, :]
            sh = 1
            while sh < nb:
                bt3[sh:, :, :] = bt3[sh:, :, :] * bt3[: nb - sh, :, :]
                sh *= 2
            ebp3[1:, :, :] = bt3[: nb - 1, :, :]
            ebp3[0:1, :, :] = jnp.ones((1, 1, n), jnp.float32)

        units = [_u_copy_in, _u_hs(1), _u_hs(2), _u_hs(4), _u_block_scan]

        send_descs = []
        unit = 0
        for d in range(N_STEPS):
            s = 1 << d
            sbuf[d, :, :] = acc[:, :]
            send = pltpu.make_async_remote_copy(
                src_ref=sbuf.at[d],
                dst_ref=rbuf.at[d],
                send_sem=send_sems.at[d],
                recv_sem=recv_sems.at[d],
                device_id=(my + s,),
                device_id_type=pl.DeviceIdType.MESH,
            )
            send_descs.append(send)

            @pl.when(my + s < N_DEV)
            def _():
                send.start()

            if unit < len(units):
                units[unit]()
                unit += 1

            @pl.when(my >= s)
            def _():
                send.wait_recv()
                acc[:, :] = acc[:, :] * rbuf[d, :, :]
                pl.semaphore_signal(
                    ack_sem, inc=1,
                    device_id=(my - s,), device_id_type=pl.DeviceIdType.MESH,
                )

        while unit < len(units):
            units[unit]()
            unit += 1

        for d in range(N_STEPS):
            s = 1 << d

            @pl.when(my + s < N_DEV)
            def _():
                send_descs[d].wait_send()
                pl.semaphore_wait(ack_sem, 1)

        acc[:, :] = acc[:, :] / tbuf[:, :]
        ebp3[:, :, :] = ebp3[:, :, :] * acc[:, :]
        out_ref[:, :] = jnp.reshape(
            s3[:, :, :] * ebp3[:, :, :], (m, n)
        )

    return pl.pallas_call(
        body,
        out_shape=jax.ShapeDtypeStruct((m, n), jnp.float32),
        in_specs=[pl.BlockSpec(memory_space=pltpu.VMEM)],
        out_specs=pl.BlockSpec(memory_space=pltpu.VMEM),
        scratch_shapes=[
            pltpu.VMEM((1, n), jnp.float32),
            pltpu.VMEM((1, n), jnp.float32),
            pltpu.VMEM((nb, B, n), jnp.float32),
            pltpu.VMEM((nb, 1, n), jnp.float32),
            pltpu.VMEM((nb, 1, n), jnp.float32),
            pltpu.VMEM((N_STEPS, 1, n), jnp.float32),
            pltpu.VMEM((N_STEPS, 1, n), jnp.float32),
            pltpu.SemaphoreType.DMA((N_STEPS,)),
            pltpu.SemaphoreType.DMA((N_STEPS,)),
            pltpu.SemaphoreType.REGULAR,
        ],
        compiler_params=pltpu.CompilerParams(collective_id=0),
    )(x)


# device time: 19778 ns/iter; 1.2719x vs baseline; 1.2478x over previous
import jax
import jax.numpy as jnp
from jax import lax
from jax.experimental import pallas as pl
from jax.experimental.pallas import tpu as pltpu

N_DEV = 32
B = 8


def kernel(x):
    m, n = x.shape
    nb = m // B

    def body(
        x_ref, out_ref, tbuf, pfx, pcopy, sbuf, s3, bt2, ebp2,
        send_sem, recv_sem, ack_sem,
    ):
        my = lax.axis_index("i")
        last = N_DEV - 1

        barrier_sem = pltpu.get_barrier_semaphore()

        @pl.when(my < last)
        def _():
            pl.semaphore_signal(
                barrier_sem, inc=1,
                device_id=(my + 1,), device_id_type=pl.DeviceIdType.MESH,
            )

        @pl.when(my > 0)
        def _():
            pl.semaphore_signal(
                barrier_sem, inc=1,
                device_id=(my - 1,), device_id_type=pl.DeviceIdType.MESH,
            )

        out_ref[:, :] = x_ref[:, :]
        size = m
        while size > 1:
            half = size // 2
            out_ref[:half, :] = out_ref[:half, :] * out_ref[half:size, :]
            size = half
        tbuf[:, :] = out_ref[0:1, :]

        @pl.when(my < last)
        def _():
            pl.semaphore_wait(barrier_sem, 1)

        @pl.when(my > 0)
        def _():
            pl.semaphore_wait(barrier_sem, 1)

        pcopy[:, :] = jnp.ones((1, n), jnp.float32)
        recv_desc = pltpu.make_async_remote_copy(
            src_ref=sbuf,
            dst_ref=pfx,
            send_sem=send_sem,
            recv_sem=recv_sem,
            device_id=(0,),
            device_id_type=pl.DeviceIdType.MESH,
        )

        @pl.when(my > 0)
        def _():
            recv_desc.wait_recv()
            pcopy[:, :] = pfx[:, :]
            pl.semaphore_signal(
                ack_sem, inc=1,
                device_id=(my - 1,), device_id_type=pl.DeviceIdType.MESH,
            )

        sbuf[:, :] = pcopy[:, :] * tbuf[:, :]
        send_desc = pltpu.make_async_remote_copy(
            src_ref=sbuf,
            dst_ref=pfx,
            send_sem=send_sem,
            recv_sem=recv_sem,
            device_id=(my + 1,),
            device_id_type=pl.DeviceIdType.MESH,
        )

        @pl.when(my < last)
        def _():
            send_desc.start()

        s3[:, :, :] = jnp.reshape(x_ref[:, :], (nb, B, n))
        for sh in (1, 2, 4):
            s3[:, sh:, :] = s3[:, sh:, :] * s3[:, : B - sh, :]

        bt2[:, :] = jnp.reshape(s3[:, B - 1 : B, :], (nb, n))
        sh = 1
        while sh < nb:
            bt2[sh:, :] = bt2[sh:, :] * bt2[: nb - sh, :]
            sh *= 2
        ebp2[1:, :] = bt2[: nb - 1, :]
        ebp2[0:1, :] = jnp.ones((1, n), jnp.float32)

        ebp2[:, :] = ebp2[:, :] * pcopy[:, :]

        out_ref[:, :] = jnp.reshape(
            s3[:, :, :] * jnp.reshape(ebp2[:, :], (nb, 1, n)), (m, n)
        )

        @pl.when(my < last)
        def _():
            send_desc.wait_send()
            pl.semaphore_wait(ack_sem, 1)

    return pl.pallas_call(
        body,
        out_shape=jax.ShapeDtypeStruct((m, n), jnp.float32),
        in_specs=[pl.BlockSpec(memory_space=pltpu.VMEM)],
        out_specs=pl.BlockSpec(memory_space=pltpu.VMEM),
        scratch_shapes=[
            pltpu.VMEM((1, n), jnp.float32),
            pltpu.VMEM((1, n), jnp.float32),
            pltpu.VMEM((1, n), jnp.float32),
            pltpu.VMEM((1, n), jnp.float32),
            pltpu.VMEM((nb, B, n), jnp.float32),
            pltpu.VMEM((nb, n), jnp.float32),
            pltpu.VMEM((nb, n), jnp.float32),
            pltpu.SemaphoreType.DMA,
            pltpu.SemaphoreType.DMA,
            pltpu.SemaphoreType.REGULAR,
        ],
        compiler_params=pltpu.CompilerParams(collective_id=0),
    )(x)


# device time: 18516 ns/iter; 1.3586x vs baseline; 1.0682x over previous
import jax
import jax.numpy as jnp
from jax import lax
from jax.experimental import pallas as pl
from jax.experimental.pallas import tpu as pltpu

N_DEV = 32
B = 8


def kernel(x):
    m, n = x.shape
    nb = m // B

    def body(
        x_ref, out_ref, pfx, pcopy, sbuf, s3, bt2, ebp2,
        send_sem, recv_sem, ack_sem,
    ):
        my = lax.axis_index("i")
        last = N_DEV - 1

        barrier_sem = pltpu.get_barrier_semaphore()

        @pl.when(my < last)
        def _():
            pl.semaphore_signal(
                barrier_sem, inc=1,
                device_id=(my + 1,), device_id_type=pl.DeviceIdType.MESH,
            )

        @pl.when(my > 0)
        def _():
            pl.semaphore_signal(
                barrier_sem, inc=1,
                device_id=(my - 1,), device_id_type=pl.DeviceIdType.MESH,
            )

        s3[:, :, :] = jnp.reshape(x_ref[:, :], (nb, B, n))
        for sh in (1, 2, 4):
            s3[:, sh:, :] = s3[:, sh:, :] * s3[:, : B - sh, :]

        bt2[:, :] = jnp.reshape(s3[:, B - 1 : B, :], (nb, n))
        sh = 1
        while sh < nb:
            bt2[sh:, :] = bt2[sh:, :] * bt2[: nb - sh, :]
            sh *= 2

        @pl.when(my < last)
        def _():
            pl.semaphore_wait(barrier_sem, 1)

        @pl.when(my > 0)
        def _():
            pl.semaphore_wait(barrier_sem, 1)

        pcopy[:, :] = jnp.ones((1, n), jnp.float32)
        recv_desc = pltpu.make_async_remote_copy(
            src_ref=sbuf,
            dst_ref=pfx,
            send_sem=send_sem,
            recv_sem=recv_sem,
            device_id=(0,),
            device_id_type=pl.DeviceIdType.MESH,
        )

        @pl.when(my > 0)
        def _():
            recv_desc.wait_recv()
            pcopy[:, :] = pfx[:, :]
            pl.semaphore_signal(
                ack_sem, inc=1,
                device_id=(my - 1,), device_id_type=pl.DeviceIdType.MESH,
            )

        sbuf[:, :] = pcopy[:, :] * bt2[nb - 1 : nb, :]
        send_desc = pltpu.make_async_remote_copy(
            src_ref=sbuf,
            dst_ref=pfx,
            send_sem=send_sem,
            recv_sem=recv_sem,
            device_id=(my + 1,),
            device_id_type=pl.DeviceIdType.MESH,
        )

        @pl.when(my < last)
        def _():
            send_desc.start()

        ebp2[1:, :] = bt2[: nb - 1, :] * pcopy[:, :]
        ebp2[0:1, :] = pcopy[:, :]

        out_ref[:, :] = jnp.reshape(
            s3[:, :, :] * jnp.reshape(ebp2[:, :], (nb, 1, n)), (m, n)
        )

        @pl.when(my < last)
        def _():
            send_desc.wait_send()
            pl.semaphore_wait(ack_sem, 1)

    return pl.pallas_call(
        body,
        out_shape=jax.ShapeDtypeStruct((m, n), jnp.float32),
        in_specs=[pl.BlockSpec(memory_space=pltpu.VMEM)],
        out_specs=pl.BlockSpec(memory_space=pltpu.VMEM),
        scratch_shapes=[
            pltpu.VMEM((1, n), jnp.float32),
            pltpu.VMEM((1, n), jnp.float32),
            pltpu.VMEM((1, n), jnp.float32),
            pltpu.VMEM((nb, B, n), jnp.float32),
            pltpu.VMEM((nb, n), jnp.float32),
            pltpu.VMEM((nb, n), jnp.float32),
            pltpu.SemaphoreType.DMA,
            pltpu.SemaphoreType.DMA,
            pltpu.SemaphoreType.REGULAR,
        ],
        compiler_params=pltpu.CompilerParams(collective_id=0),
    )(x)
